# baseline (speedup 1.0000x reference)
import functools

import jax
import jax.numpy as jnp
from jax import lax
from jax.experimental import pallas as pl
from jax.experimental.pallas import tpu as pltpu

_MIB = 1024 * 1024


def _se_kernel(x_ref, w1_ref, b1_ref, w2t_ref, b2_ref, o_ref,
               cache_ref, acc_ref, g_ref, *, n_ta, tile_a, tile_b, inv_hw):
    t = pl.program_id(1)

    @pl.when(t == 0)
    def _():
        acc_ref[...] = jnp.zeros_like(acc_ref)

    @pl.when(t < n_ta)
    def _():
        xv = x_ref[...]
        acc_ref[...] += jnp.sum(xv, axis=0)
        cache_ref[pl.ds(t * tile_a, tile_a)] = xv.astype(cache_ref.dtype)

    @pl.when(t == n_ta)
    def _():
        avg = acc_ref[...] * inv_hw
        h = lax.dot_general(avg, w1_ref[...], (((1,), (1,)), ((), ())),
                            preferred_element_type=jnp.float32) + b1_ref[...]
        h = jnp.where(h >= 0.0, h, 0.2 * h)
        s = jnp.dot(h, w2t_ref[...],
                    preferred_element_type=jnp.float32) + b2_ref[...]
        g_ref[...] = jax.nn.sigmoid(s)

    @pl.when(t >= n_ta)
    def _():
        xv = cache_ref[pl.ds((t - n_ta) * tile_b, tile_b)].astype(jnp.float32)
        o_ref[...] = (xv * g_ref[...]).astype(o_ref.dtype)


def _pick_hw_tile(hw, n, c, itemsize, budget_bytes):
    best = 1
    for t in range(1, hw + 1):
        if hw % t:
            continue
        if t * n * c * itemsize <= budget_bytes:
            best = t
    return best


@jax.jit
def _se_forward(x_nchw, w1, b1, w2, b2):
    N, C, H, W = x_nchw.shape
    Cr = w1.shape[0]
    HW = H * W
    Nh = N // 2

    xs = jnp.transpose(x_nchw, (2, 3, 0, 1)).reshape(HW, N, C)
    w2t = w2.T
    b1r = b1.reshape(1, Cr)
    b2r = b2.reshape(1, C)

    Ta = _pick_hw_tile(HW, Nh, C, x_nchw.dtype.itemsize, 10 * _MIB)
    Tb = _pick_hw_tile(HW, Nh, C, x_nchw.dtype.itemsize, 6 * _MIB)
    nTa = HW // Ta
    nTb = HW // Tb

    out = pl.pallas_call(
        functools.partial(_se_kernel, n_ta=nTa, tile_a=Ta, tile_b=Tb,
                          inv_hw=1.0 / HW),
        out_shape=jax.ShapeDtypeStruct((HW, N, C), x_nchw.dtype),
        grid=(2, nTa + nTb),
        in_specs=[
            pl.BlockSpec((Ta, Nh, C),
                         lambda i, t: (jnp.minimum(t, nTa - 1), i, 0)),
            pl.BlockSpec((Cr, C), lambda i, t: (0, 0)),
            pl.BlockSpec((1, Cr), lambda i, t: (0, 0)),
            pl.BlockSpec((Cr, C), lambda i, t: (0, 0)),
            pl.BlockSpec((1, C), lambda i, t: (0, 0)),
        ],
        out_specs=pl.BlockSpec((Tb, Nh, C),
                               lambda i, t: (jnp.maximum(t - nTa, 0), i, 0)),
        scratch_shapes=[
            pltpu.VMEM((HW, Nh, C), jnp.bfloat16),
            pltpu.VMEM((Nh, C), jnp.float32),
            pltpu.VMEM((Nh, C), jnp.float32),
        ],
        compiler_params=pltpu.CompilerParams(
            dimension_semantics=("parallel", "arbitrary"),
            vmem_limit_bytes=58 * _MIB),
    )(xs, w1, b1r, w2t, b2r)

    return out.reshape(H, W, N, C).transpose(2, 3, 0, 1)


def kernel(x_nchw, w1, b1, w2, b2):
    return _se_forward(x_nchw, w1, b1, w2, b2)
